# baseline (speedup 1.0000x reference)
import jax
import jax.numpy as jnp
from jax import lax
from jax.experimental import pallas as pl
from jax.experimental.pallas import tpu as pltpu
from jax.experimental.pallas import tpu_sc as plsc

NUM_EMB = 1000000
DIM = 64
PDIM = 128
NC = 2
NS = 16
NW = NC * NS

B_TOTAL = 4096 * 200
B_PER_W = B_TOTAL // NW
CHUNK = 128
N_CHUNKS = B_PER_W // CHUNK
NBUF = 5


def _emb_body(tok_hbm, weight_hbm, out_hbm, rows_v, *rest):
    idx_bufs = rest[:NBUF]
    gsem = rest[NBUF:]
    wid = lax.axis_index("s") * NC + lax.axis_index("c")
    base = wid * B_PER_W

    rows = [rows_v.at[b] for b in range(NBUF)]

    def start_gather(c, buf):
        pltpu.sync_copy(tok_hbm.at[pl.ds(base + c * CHUNK, CHUNK)], idx_bufs[buf])
        pltpu.async_copy(weight_hbm.at[idx_bufs[buf]], rows[buf], gsem[buf])

    def wait_gather(buf):
        pltpu.make_async_copy(
            weight_hbm.at[idx_bufs[buf]], rows[buf], gsem[buf]
        ).wait()

    def write_out(c, buf):
        pltpu.sync_copy(
            rows[buf],
            out_hbm.at[pl.ds(base + c * CHUNK, CHUNK), pl.ds(0, DIM)],
        )

    for b in range(NBUF):
        start_gather(b, b)

    def group(g, _):
        for b in range(NBUF):
            c = g * NBUF + b
            wait_gather(b)
            write_out(c, b)
            start_gather(c + NBUF, b)
        return _

    lax.fori_loop(0, (N_CHUNKS - NBUF) // NBUF, group, 0)

    for b in range(NBUF):
        c = N_CHUNKS - NBUF + b
        wait_gather(b)
        write_out(c, b)


@jax.jit
def kernel(token_ids, weight):
    tokf = token_ids.reshape(B_TOTAL)
    mesh = plsc.VectorSubcoreMesh(core_axis_name="c", subcore_axis_name="s")
    outp = pl.kernel(
        _emb_body,
        out_type=jax.ShapeDtypeStruct((B_TOTAL, PDIM), jnp.float32),
        mesh=mesh,
        scratch_types=[
            pltpu.VMEM((NBUF, CHUNK, DIM), jnp.float32),
        ] + [pltpu.VMEM((CHUNK,), jnp.int32)] * NBUF
          + [pltpu.SemaphoreType.DMA] * NBUF,
        compiler_params=pltpu.CompilerParams(use_tc_tiling_on_sc=False),
    )(tokf, weight)
    return outp.reshape(4096, 200, PDIM)[..., :DIM]


# speedup vs baseline: 1.0149x; 1.0149x over previous
import jax
import jax.numpy as jnp
from jax import lax
from jax.experimental import pallas as pl
from jax.experimental.pallas import tpu as pltpu
from jax.experimental.pallas import tpu_sc as plsc

NUM_EMB = 1000000
DIM = 64
PDIM = 128
NC = 2
NS = 16
NW = NC * NS

B_TOTAL = 4096 * 200
B_PER_W = B_TOTAL // NW
SUB = 128
NSUB = 2
CHUNK = SUB * NSUB
N_CHUNKS = B_PER_W // CHUNK
NBUF = 4


def _emb_body(tok_hbm, weight_hbm, out_hbm, rows_v, *rest):
    idx_bufs = rest[:NBUF * NSUB]
    gsem = rest[NBUF * NSUB:]
    wid = lax.axis_index("s") * NC + lax.axis_index("c")
    base = wid * B_PER_W

    def start_gather(c, buf):
        for s in range(NSUB):
            ib = idx_bufs[buf * NSUB + s]
            pltpu.sync_copy(
                tok_hbm.at[pl.ds(base + c * CHUNK + s * SUB, SUB)], ib
            )
            pltpu.async_copy(
                weight_hbm.at[ib],
                rows_v.at[buf, pl.ds(s * SUB, SUB)],
                gsem[buf],
            )

    def wait_gather(buf):
        for s in range(NSUB):
            pltpu.make_async_copy(
                weight_hbm.at[idx_bufs[buf * NSUB + s]],
                rows_v.at[buf, pl.ds(s * SUB, SUB)],
                gsem[buf],
            ).wait()

    def write_out(c, buf):
        pltpu.sync_copy(
            rows_v.at[buf],
            out_hbm.at[pl.ds(base + c * CHUNK, CHUNK), pl.ds(0, DIM)],
        )

    for b in range(NBUF):
        start_gather(b, b)

    def group(g, _):
        for b in range(NBUF):
            c = g * NBUF + b
            wait_gather(b)
            write_out(c, b)
            start_gather(c + NBUF, b)
        return _

    lax.fori_loop(0, (N_CHUNKS - NBUF) // NBUF, group, 0)

    for b in range(NBUF):
        c = N_CHUNKS - NBUF + b
        wait_gather(b)
        write_out(c, b)


@jax.jit
def kernel(token_ids, weight):
    tokf = token_ids.reshape(B_TOTAL)
    mesh = plsc.VectorSubcoreMesh(core_axis_name="c", subcore_axis_name="s")
    outp = pl.kernel(
        _emb_body,
        out_type=jax.ShapeDtypeStruct((B_TOTAL, PDIM), jnp.float32),
        mesh=mesh,
        scratch_types=[
            pltpu.VMEM((NBUF, CHUNK, DIM), jnp.float32),
        ] + [pltpu.VMEM((SUB,), jnp.int32)] * (NBUF * NSUB)
          + [pltpu.SemaphoreType.DMA] * NBUF,
        compiler_params=pltpu.CompilerParams(use_tc_tiling_on_sc=False),
    )(tokf, weight)
    return outp.reshape(4096, 200, PDIM)[..., :DIM]
